# Initial kernel scaffold; baseline (speedup 1.0000x reference)
import functools

import jax
import jax.numpy as jnp
from jax import lax
from jax.experimental import pallas as pl
from jax.experimental.pallas import tpu as pltpu
from jax.experimental.pallas import tpu_sc as plsc

N = 10000
NP = 10240
D = 128
G = 64
OUT = 10
E = 320000
CH = 128
NCHUNK = 80
NW = 32
EP = CH * NCHUNK * NW
RPT = NP // 16

_f32 = jnp.float32
_mesh = plsc.VectorSubcoreMesh(core_axis_name="c", subcore_axis_name="s")


@functools.partial(
    pl.kernel,
    mesh=_mesh,
    out_type=jax.ShapeDtypeStruct((2, NP), _f32),
    scratch_types=[
        pltpu.VMEM((NCHUNK, CH), jnp.int32),
        pltpu.VMEM((CH,), _f32),
        pltpu.VMEM((RPT,), _f32),
        pltpu.VMEM_SHARED((NP,), _f32),
    ],
)
def _deg(dst_hbm, out_hbm, dst_idx, ones_v, zeros_v, counts):
    c = lax.axis_index("c")
    s = lax.axis_index("s")
    w = c * 16 + s
    pltpu.sync_copy(dst_hbm.at[w], dst_idx)

    def _fill(i, _):
        zeros_v[pl.ds(i * 16, 16)] = jnp.zeros((16,), _f32)
        return 0
    lax.fori_loop(0, RPT // 16, _fill, 0)
    for k in range(CH // 16):
        ones_v[pl.ds(k * 16, 16)] = jnp.ones((16,), _f32)

    pltpu.sync_copy(zeros_v, counts.at[pl.ds(s * RPT, RPT)])
    plsc.subcore_barrier()

    def _chunk(j, _):
        pltpu.sync_copy(ones_v, counts.at[dst_idx.at[j]], add=True)
        return 0
    lax.fori_loop(0, NCHUNK, _chunk, 0)

    plsc.subcore_barrier()
    pltpu.sync_copy(counts.at[pl.ds(s * RPT, RPT)],
                    out_hbm.at[c, pl.ds(s * RPT, RPT)])


@functools.partial(
    pl.kernel,
    mesh=_mesh,
    out_type=jax.ShapeDtypeStruct((2, NP, D), _f32),
    scratch_types=[
        pltpu.VMEM((NCHUNK, CH), jnp.int32),
        pltpu.VMEM((NCHUNK, CH), jnp.int32),
        pltpu.VMEM((CH, D), _f32),
        pltpu.VMEM((CH, D), _f32),
        pltpu.VMEM((CH, D), _f32),
        pltpu.VMEM_SHARED((NP, D), _f32),
        pltpu.SemaphoreType.DMA,
        pltpu.SemaphoreType.DMA,
    ],
)
def _agg(gs_hbm, src_hbm, dst_hbm, out_hbm,
         src_idx, dst_idx, r0, r1, zb, acc, sem0, sem1):
    c = lax.axis_index("c")
    s = lax.axis_index("s")
    w = c * 16 + s
    pltpu.sync_copy(src_hbm.at[w], src_idx)
    pltpu.sync_copy(dst_hbm.at[w], dst_idx)

    def _zrow(i, _):
        for k in range(D // 16):
            zb[i, pl.ds(k * 16, 16)] = jnp.zeros((16,), _f32)
        return 0
    lax.fori_loop(0, CH, _zrow, 0)
    for z in range(RPT // CH):
        pltpu.sync_copy(zb, acc.at[pl.ds(s * RPT + z * CH, CH)])
    plsc.subcore_barrier()

    pltpu.async_copy(gs_hbm.at[src_idx.at[0]], r0, sem0)

    def _step(g, _):
        j0 = 2 * g
        pltpu.make_async_copy(gs_hbm.at[src_idx.at[j0]], r0, sem0).wait()
        pltpu.async_copy(gs_hbm.at[src_idx.at[j0 + 1]], r1, sem1)
        pltpu.sync_copy(r0, acc.at[dst_idx.at[j0]], add=True)

        @pl.when(g < NCHUNK // 2 - 1)
        def _():
            pltpu.async_copy(gs_hbm.at[src_idx.at[j0 + 2]], r0, sem0)

        pltpu.make_async_copy(gs_hbm.at[src_idx.at[j0 + 1]], r1, sem1).wait()
        pltpu.sync_copy(r1, acc.at[dst_idx.at[j0 + 1]], add=True)
        return 0
    lax.fori_loop(0, NCHUNK // 2, _step, 0)

    plsc.subcore_barrier()
    pltpu.sync_copy(acc.at[pl.ds(s * RPT, RPT)],
                    out_hbm.at[c, pl.ds(s * RPT, RPT)])


def _k1_body(x_ref, w_ref, c0_ref, c1_ref, gs_ref, dinv_ref):
    deg = c0_ref[...] + c1_ref[...] + 1.0
    dinv = lax.rsqrt(deg)
    g = jnp.dot(x_ref[...], w_ref[...], preferred_element_type=_f32)
    gs_ref[...] = g * dinv
    dinv_ref[...] = dinv


_k1 = pl.pallas_call(
    _k1_body,
    out_shape=(jax.ShapeDtypeStruct((NP, D), _f32),
               jax.ShapeDtypeStruct((NP, 1), _f32)),
)


def _kmid_body(s_ref, gsp_ref, dinv_ref, b_ref, w_ref, gs_ref):
    dinv = dinv_ref[...]
    h = jnp.maximum((s_ref[0] + s_ref[1] + gsp_ref[...]) * dinv + b_ref[...],
                    0.0)
    gs_ref[...] = jnp.dot(h, w_ref[...], preferred_element_type=_f32) * dinv


_kmid = pl.pallas_call(
    _kmid_body,
    out_shape=jax.ShapeDtypeStruct((NP, D), _f32),
)


def _klast_body(s_ref, gsp_ref, dinv_ref, b_ref, batch_ref,
                wl1_ref, bl1_ref, wl2_ref, bl2_ref, out_ref):
    h = jnp.maximum(
        (s_ref[0] + s_ref[1] + gsp_ref[...]) * dinv_ref[...] + b_ref[...],
        0.0)
    io = lax.broadcasted_iota(jnp.int32, (NP, G), 1)
    onehot = (batch_ref[...] == io).astype(_f32)
    pooled = lax.dot_general(onehot, h, (((0,), (0,)), ((), ())),
                             preferred_element_type=_f32)
    cnt = lax.dot_general(onehot, jnp.ones((NP, 1), _f32),
                          (((0,), (0,)), ((), ())),
                          preferred_element_type=_f32)
    pooled = pooled / jnp.maximum(cnt, 1.0)
    h2 = jnp.maximum(
        jnp.dot(pooled, wl1_ref[...], preferred_element_type=_f32)
        + bl1_ref[...], 0.0)
    logits = (jnp.dot(h2, wl2_ref[...], preferred_element_type=_f32)
              + bl2_ref[...])
    m = jnp.max(logits, axis=-1, keepdims=True)
    sh = logits - m
    out_ref[...] = sh - jnp.log(jnp.sum(jnp.exp(sh), axis=-1, keepdims=True))


_klast = pl.pallas_call(
    _klast_body,
    out_shape=jax.ShapeDtypeStruct((G, OUT), _f32),
)


def kernel(x, edge_index, batch, W1, b1, W2, b2, W3, b3, Wl1, bl1, Wl2, bl2):
    padn = EP - E
    ar = jnp.arange(padn, dtype=jnp.int32)
    pad_src = (ar * 97) % N
    pad_dst = N + (ar % (NP - N))
    srcs = jnp.concatenate([edge_index[0], pad_src]).reshape(NW, NCHUNK, CH)
    dsts = jnp.concatenate([edge_index[1], pad_dst]).reshape(NW, NCHUNK, CH)
    xp = jnp.pad(x, ((0, NP - N), (0, 0)))
    batchp = jnp.pad(batch, (0, NP - N), constant_values=G).reshape(NP, 1)

    counts = _deg(dsts)
    c0 = counts[0].reshape(NP, 1)
    c1 = counts[1].reshape(NP, 1)

    gs1, dinv = _k1(xp, W1, c0, c1)
    s1 = _agg(gs1, srcs, dsts)
    gs2 = _kmid(s1, gs1, dinv, b1.reshape(1, D), W2)
    s2 = _agg(gs2, srcs, dsts)
    gs3 = _kmid(s2, gs2, dinv, b2.reshape(1, D), W3)
    s3 = _agg(gs3, srcs, dsts)
    return _klast(s3, gs3, dinv, b3.reshape(1, D), batchp,
                  Wl1, bl1.reshape(1, D), Wl2, bl2.reshape(1, OUT))


# speedup vs baseline: 28.1500x; 28.1500x over previous
import functools

import jax
import jax.numpy as jnp
from jax import lax
from jax.experimental import pallas as pl
from jax.experimental.pallas import tpu as pltpu
from jax.experimental.pallas import tpu_sc as plsc

N = 10000
NP = 10240
D = 128
G = 64
OUT = 10
E = 320000
CH = 128
NCHUNK = 80
ROUNDS = 2
CPR = NCHUNK // ROUNDS
NW = 32
EP = CH * NCHUNK * NW
RPT = NP // 16

_f32 = jnp.float32
_mesh = plsc.VectorSubcoreMesh(core_axis_name="c", subcore_axis_name="s")


@functools.partial(
    pl.kernel,
    mesh=_mesh,
    out_type=jax.ShapeDtypeStruct((2, NP), _f32),
    scratch_types=[
        pltpu.VMEM((NCHUNK, CH), jnp.int32),
        pltpu.VMEM((CH,), _f32),
        pltpu.VMEM((RPT,), _f32),
        pltpu.VMEM_SHARED((NP,), _f32),
    ],
)
def _deg(dst_hbm, out_hbm, dst_idx, ones_v, zeros_v, counts):
    c = lax.axis_index("c")
    s = lax.axis_index("s")
    w = c * 16 + s
    pltpu.sync_copy(dst_hbm.at[w], dst_idx)

    def _fill(i, _):
        zeros_v[pl.ds(i * 16, 16)] = jnp.zeros((16,), _f32)
        return 0
    lax.fori_loop(0, RPT // 16, _fill, 0)
    for k in range(CH // 16):
        ones_v[pl.ds(k * 16, 16)] = jnp.ones((16,), _f32)

    pltpu.sync_copy(zeros_v, counts.at[pl.ds(s * RPT, RPT)])
    plsc.subcore_barrier()

    def _chunk(j, _):
        pltpu.sync_copy(ones_v, counts.at[dst_idx.at[j]], add=True)
        return 0
    lax.fori_loop(0, NCHUNK, _chunk, 0)

    plsc.subcore_barrier()
    pltpu.sync_copy(counts.at[pl.ds(s * RPT, RPT)],
                    out_hbm.at[c, pl.ds(s * RPT, RPT)])


@functools.partial(
    pl.kernel,
    mesh=_mesh,
    out_type=jax.ShapeDtypeStruct((2, NP, D), _f32),
    scratch_types=[
        pltpu.VMEM((CPR, CH), jnp.int32),
        pltpu.VMEM((CPR, CH), jnp.int32),
        pltpu.VMEM((CH, D), _f32),
        pltpu.VMEM((CH, D), _f32),
        pltpu.VMEM_SHARED((NP, D), _f32),
        pltpu.SemaphoreType.DMA,
        pltpu.SemaphoreType.DMA,
    ],
)
def _agg(gs_hbm, src_hbm, dst_hbm, out_hbm,
         src_idx, dst_idx, r0, r1, acc, sem0, sem1):
    c = lax.axis_index("c")
    s = lax.axis_index("s")
    w = c * 16 + s

    def _zrow(i, _):
        for k in range(D // 16):
            r0[i, pl.ds(k * 16, 16)] = jnp.zeros((16,), _f32)
        return 0
    lax.fori_loop(0, CH, _zrow, 0)
    for z in range(RPT // CH):
        pltpu.sync_copy(r0, acc.at[pl.ds(s * RPT + z * CH, CH)])
    plsc.subcore_barrier()

    def _round(r, _):
        pltpu.sync_copy(src_hbm.at[w, pl.ds(r * CPR, CPR)], src_idx)
        pltpu.sync_copy(dst_hbm.at[w, pl.ds(r * CPR, CPR)], dst_idx)
        pltpu.async_copy(gs_hbm.at[src_idx.at[0]], r0, sem0)

        def _step(g, _):
            j0 = 2 * g
            pltpu.make_async_copy(gs_hbm.at[src_idx.at[j0]], r0, sem0).wait()
            pltpu.async_copy(gs_hbm.at[src_idx.at[j0 + 1]], r1, sem1)
            pltpu.sync_copy(r0, acc.at[dst_idx.at[j0]], add=True)

            @pl.when(g < CPR // 2 - 1)
            def _():
                pltpu.async_copy(gs_hbm.at[src_idx.at[j0 + 2]], r0, sem0)

            pltpu.make_async_copy(gs_hbm.at[src_idx.at[j0 + 1]], r1,
                                  sem1).wait()
            pltpu.sync_copy(r1, acc.at[dst_idx.at[j0 + 1]], add=True)
            return 0
        lax.fori_loop(0, CPR // 2, _step, 0)
        return 0
    lax.fori_loop(0, ROUNDS, _round, 0)

    plsc.subcore_barrier()
    pltpu.sync_copy(acc.at[pl.ds(s * RPT, RPT)],
                    out_hbm.at[c, pl.ds(s * RPT, RPT)])


def _k1_body(x_ref, w_ref, c0_ref, c1_ref, gs_ref, dinv_ref):
    deg = c0_ref[...] + c1_ref[...] + 1.0
    dinv = lax.rsqrt(deg)
    g = jnp.dot(x_ref[...], w_ref[...], preferred_element_type=_f32)
    gs_ref[...] = g * dinv
    dinv_ref[...] = dinv


_k1 = pl.pallas_call(
    _k1_body,
    out_shape=(jax.ShapeDtypeStruct((NP, D), _f32),
               jax.ShapeDtypeStruct((NP, 1), _f32)),
)


def _kmid_body(s_ref, gsp_ref, dinv_ref, b_ref, w_ref, gs_ref):
    dinv = dinv_ref[...]
    h = jnp.maximum((s_ref[0] + s_ref[1] + gsp_ref[...]) * dinv + b_ref[...],
                    0.0)
    gs_ref[...] = jnp.dot(h, w_ref[...], preferred_element_type=_f32) * dinv


_kmid = pl.pallas_call(
    _kmid_body,
    out_shape=jax.ShapeDtypeStruct((NP, D), _f32),
)


def _klast_body(s_ref, gsp_ref, dinv_ref, b_ref, batch_ref,
                wl1_ref, bl1_ref, wl2_ref, bl2_ref, out_ref):
    h = jnp.maximum(
        (s_ref[0] + s_ref[1] + gsp_ref[...]) * dinv_ref[...] + b_ref[...],
        0.0)
    io = lax.broadcasted_iota(jnp.int32, (NP, G), 1)
    onehot = (batch_ref[...] == io).astype(_f32)
    pooled = lax.dot_general(onehot, h, (((0,), (0,)), ((), ())),
                             preferred_element_type=_f32)
    cnt = lax.dot_general(onehot, jnp.ones((NP, 1), _f32),
                          (((0,), (0,)), ((), ())),
                          preferred_element_type=_f32)
    pooled = pooled / jnp.maximum(cnt, 1.0)
    h2 = jnp.maximum(
        jnp.dot(pooled, wl1_ref[...], preferred_element_type=_f32)
        + bl1_ref[...], 0.0)
    logits = (jnp.dot(h2, wl2_ref[...], preferred_element_type=_f32)
              + bl2_ref[...])
    m = jnp.max(logits, axis=-1, keepdims=True)
    sh = logits - m
    out_ref[...] = sh - jnp.log(jnp.sum(jnp.exp(sh), axis=-1, keepdims=True))


_klast = pl.pallas_call(
    _klast_body,
    out_shape=jax.ShapeDtypeStruct((G, OUT), _f32),
)


def kernel(x, edge_index, batch, W1, b1, W2, b2, W3, b3, Wl1, bl1, Wl2, bl2):
    padn = EP - E
    ar = jnp.arange(padn, dtype=jnp.int32)
    pad_src = (ar * 97) % N
    pad_dst = N + (ar % (NP - N))
    srcs = jnp.concatenate([edge_index[0], pad_src]).reshape(NW, NCHUNK, CH)
    dsts = jnp.concatenate([edge_index[1], pad_dst]).reshape(NW, NCHUNK, CH)
    xp = jnp.pad(x, ((0, NP - N), (0, 0)))
    batchp = jnp.pad(batch, (0, NP - N), constant_values=G).reshape(NP, 1)

    counts = _deg(dsts)
    c0 = counts[0].reshape(NP, 1)
    c1 = counts[1].reshape(NP, 1)

    gs1, dinv = _k1(xp, W1, c0, c1)
    s1 = _agg(gs1, srcs, dsts)
    gs2 = _kmid(s1, gs1, dinv, b1.reshape(1, D), W2)
    s2 = _agg(gs2, srcs, dsts)
    gs3 = _kmid(s2, gs2, dinv, b2.reshape(1, D), W3)
    s3 = _agg(gs3, srcs, dsts)
    return _klast(s3, gs3, dinv, b3.reshape(1, D), batchp,
                  Wl1, bl1.reshape(1, D), Wl2, bl2.reshape(1, OUT))
